# baseline (speedup 1.0000x reference)
import functools

import jax
import jax.numpy as jnp
import numpy as np
from jax import lax
from jax.experimental import pallas as pl
from jax.experimental.pallas import tpu as pltpu
from jax.experimental.pallas import tpu_sc as plsc

B, C, N = 2, 1024, 4096
GROUP_SIZES = (128, 128, 256, 512)
GROUP_STARTS = (0, 128, 256, 512)

NC, NS = 2, 16
NW = NC * NS
ROWS_PER_LEAF_PER_W = tuple(2 * g // NW for g in GROUP_SIZES)
IDX_PER_W = 2 * C // NW
CHUNK = 8
NBUF = 3

_RM = np.zeros(2 * C, dtype=np.int32)
_OM = np.zeros(2 * C, dtype=np.int32)
for _w in range(NW):
    _l = 0
    for _g, _s0, _rpw in zip(GROUP_SIZES, GROUP_STARTS, ROWS_PER_LEAF_PER_W):
        for _r in range(_rpw):
            _flat = _w * _rpw + _r
            _b, _j = divmod(_flat, _g)
            _RM[_w * IDX_PER_W + _l] = _s0 + _j
            _OM[_w * IDX_PER_W + _l] = _b * C
            _l += 1


def _energy_body(q_ref, e_ref):
    x = jnp.abs(q_ref[...])
    acc = x[..., 0:128]
    for j in range(1, 32):
        acc = acc + x[..., j * 128:(j + 1) * 128]
    s = acc[..., 0:8]
    for v in range(1, 16):
        s = s + acc[..., 8 * v:8 * v + 8]
    lane = [s[..., i:i + 1] for i in range(8)]
    tot = ((lane[0] + lane[4]) + (lane[2] + lane[6])) + \
          ((lane[1] + lane[5]) + (lane[3] + lane[7]))
    inv = np.float32(1.0 / N)
    e_ref[...] = (tot[0] * inv + tot[1] * inv) * np.float32(0.5)


def _energy(q):
    return pl.pallas_call(
        _energy_body,
        grid=(8,),
        in_specs=[pl.BlockSpec((2, 128, 4096), lambda i: (0, i, 0))],
        out_specs=pl.BlockSpec((128, 1), lambda i: (i, 0)),
        out_shape=jax.ShapeDtypeStruct((C, 1), jnp.float32),
    )(q)


def _rank_body(e_ref, rm_ref, om_ref, i0_ref, i1_ref, i2_ref, i3_ref,
               gidx_ref):
    e = e_ref[...]
    e_lanes = jnp.broadcast_to(e, (C, C))
    e_rows = lax.broadcast_in_dim(e.reshape(C), (C, C), (0,))
    ii = lax.broadcasted_iota(jnp.int32, (C, C), 0)
    jj = lax.broadcasted_iota(jnp.int32, (C, C), 1)
    before = (e_rows > e_lanes) | ((e_rows == e_lanes) & (ii < jj))
    rank = jnp.sum(before.astype(jnp.int32), axis=0)
    rank_lanes = rank.reshape(1, C)

    rr = lax.broadcasted_iota(jnp.int32, (C, C), 0)
    onehot = (jnp.broadcast_to(rank_lanes, (C, C)) == rr).astype(jnp.int32)
    sorted_idx = jnp.sum(onehot * jj, axis=1)
    for ref, g, s0 in zip((i0_ref, i1_ref, i2_ref, i3_ref),
                          GROUP_SIZES, GROUP_STARTS):
        ref[...] = lax.slice(sorted_idx, (s0,), (s0 + g,)).reshape(g, 1)

    rm = rm_ref[...]
    om = om_ref[...]
    jj2 = lax.broadcasted_iota(jnp.int32, (2 * C, C), 1)
    oh2 = (jnp.broadcast_to(rank_lanes, (2 * C, C)) == rm).astype(jnp.int32)
    gidx_ref[...] = (jnp.sum(oh2 * jj2, axis=1).reshape(2 * C, 1) + om)


def _sort_and_index(e):
    return pl.pallas_call(
        _rank_body,
        out_shape=tuple(jax.ShapeDtypeStruct((g, 1), jnp.int32)
                        for g in GROUP_SIZES)
                  + (jax.ShapeDtypeStruct((2 * C, 1), jnp.int32),),
    )(e.reshape(1, C), jnp.asarray(_RM).reshape(2 * C, 1),
      jnp.asarray(_OM).reshape(2 * C, 1))


def _gather_body(q_hbm, k_hbm, v_hbm, gidx_hbm, *refs):
    outs = (refs[0:4], refs[4:8], refs[8:12])
    idx_v = refs[12]
    bufs = refs[13:13 + NBUF]
    gsems = refs[13 + NBUF:13 + 2 * NBUF]
    wsems = refs[13 + 2 * NBUF:13 + 3 * NBUF]
    tables = (q_hbm, k_hbm, v_hbm)

    wid = lax.axis_index("s") * NC + lax.axis_index("c")
    pltpu.sync_copy(gidx_hbm.at[pl.ds(wid * IDX_PER_W, IDX_PER_W)], idx_v)

    sched = []
    for t in range(3):
        off = 0
        for g in range(4):
            rpw = ROWS_PER_LEAF_PER_W[g]
            for c in range(rpw // CHUNK):
                sched.append((t, g, off + c * CHUNK, c))
            off += rpw
    S = len(sched)

    def gather(s):
        t, g, ioff, c = sched[s]
        return pltpu.async_copy(
            tables[t].at[idx_v.at[pl.ds(ioff, CHUNK)]],
            bufs[s % NBUF], gsems[s % NBUF])

    def write(s):
        t, g, ioff, c = sched[s]
        row0 = wid * ROWS_PER_LEAF_PER_W[g] + c * CHUNK
        return pltpu.async_copy(
            bufs[s % NBUF], outs[t][g].at[pl.ds(row0, CHUNK)],
            wsems[s % NBUF])

    gh = [None] * S
    wh = [None] * S
    for s in range(min(2, S)):
        gh[s] = gather(s)
    for s in range(S):
        gh[s].wait()
        wh[s] = write(s)
        if s - 1 >= 0:
            wh[s - 1].wait()
        if s + 2 < S:
            gh[s + 2] = gather(s + 2)
    wh[S - 1].wait()


def kernel(q, k, v):
    energy = _energy(q).reshape(C)
    i0, i1, i2, i3, gidx = _sort_and_index(energy)
    gidx = gidx.reshape(2 * C)
    idx_groups = [i.reshape(-1) for i in (i0, i1, i2, i3)]

    out_type = [jax.ShapeDtypeStruct((2 * g, N), jnp.float32)
                for g in GROUP_SIZES] * 3

    gather_call = functools.partial(
        pl.kernel,
        mesh=plsc.VectorSubcoreMesh(core_axis_name="c", subcore_axis_name="s"),
        out_type=out_type,
        scratch_types=(
            [pltpu.VMEM((IDX_PER_W,), jnp.int32)]
            + [pltpu.VMEM((CHUNK, N), jnp.float32)] * NBUF
            + [pltpu.SemaphoreType.DMA] * (2 * NBUF)
        ),
    )(_gather_body)

    flat = gather_call(q.reshape(B * C, N), k.reshape(B * C, N),
                       v.reshape(B * C, N), gidx)
    q_groups = [flat[i].reshape(B, g, N) for i, g in enumerate(GROUP_SIZES)]
    k_groups = [flat[4 + i].reshape(B, g, N) for i, g in enumerate(GROUP_SIZES)]
    v_groups = [flat[8 + i].reshape(B, g, N) for i, g in enumerate(GROUP_SIZES)]
    return tuple(q_groups) + tuple(k_groups) + tuple(v_groups) + tuple(idx_groups)


# speedup vs baseline: 1.0427x; 1.0427x over previous
import functools

import jax
import jax.numpy as jnp
import numpy as np
from jax import lax
from jax.experimental import pallas as pl
from jax.experimental.pallas import tpu as pltpu
from jax.experimental.pallas import tpu_sc as plsc

B, C, N = 2, 1024, 4096
GROUP_SIZES = (128, 128, 256, 512)
GROUP_STARTS = (0, 128, 256, 512)

NC, NS = 2, 16
NW = NC * NS
ROWS_PER_LEAF_PER_W = tuple(2 * g // NW for g in GROUP_SIZES)
IDX_PER_W = 2 * C // NW
CHUNK = 8
NBUF = 3

_RM = np.zeros(2 * C, dtype=np.int32)
_OM = np.zeros(2 * C, dtype=np.int32)
for _w in range(NW):
    _l = 0
    for _g, _s0, _rpw in zip(GROUP_SIZES, GROUP_STARTS, ROWS_PER_LEAF_PER_W):
        for _r in range(_rpw):
            _flat = _w * _rpw + _r
            _b, _j = divmod(_flat, _g)
            _RM[_w * IDX_PER_W + _l] = _s0 + _j
            _OM[_w * IDX_PER_W + _l] = _b * C
            _l += 1


def _energy_body(q_ref, e_ref):
    x = jnp.abs(q_ref[...])
    acc = x[..., 0:128]
    for j in range(1, 32):
        acc = acc + x[..., j * 128:(j + 1) * 128]
    accT = lax.transpose(acc, (0, 2, 1))
    s = accT[:, 0:8, :]
    for v in range(1, 16):
        s = s + accT[:, 8 * v:8 * v + 8, :]
    u = s[:, 0:4, :] + s[:, 4:8, :]
    p = u[:, 0:2, :] + u[:, 2:4, :]
    tot = p[:, 0:1, :] + p[:, 1:2, :]
    inv = np.float32(1.0 / N)
    e_ref[...] = (tot[0:1, :, :] * inv + tot[1:2, :, :] * inv) * np.float32(0.5)


def _energy(q):
    return pl.pallas_call(
        _energy_body,
        grid=(8,),
        in_specs=[pl.BlockSpec((2, 128, 4096), lambda i: (0, i, 0))],
        out_specs=pl.BlockSpec((1, 1, 128), lambda i: (i, 0, 0)),
        out_shape=jax.ShapeDtypeStruct((8, 1, 128), jnp.float32),
    )(q)


def _rank_body(e_ref, rm_ref, om_ref, i0_ref, i1_ref, i2_ref, i3_ref,
               gidx_ref):
    e = e_ref[...]
    e_lanes = jnp.broadcast_to(e, (C, C))
    e_rows = lax.broadcast_in_dim(e.reshape(C), (C, C), (0,))
    ii = lax.broadcasted_iota(jnp.int32, (C, C), 0)
    jj = lax.broadcasted_iota(jnp.int32, (C, C), 1)
    before = (e_rows > e_lanes) | ((e_rows == e_lanes) & (ii < jj))
    rank = jnp.sum(before.astype(jnp.int32), axis=0)
    rank_lanes = rank.reshape(1, C)

    rr = lax.broadcasted_iota(jnp.int32, (C, C), 0)
    onehot = (jnp.broadcast_to(rank_lanes, (C, C)) == rr).astype(jnp.int32)
    sorted_idx = jnp.sum(onehot * jj, axis=1)
    for ref, g, s0 in zip((i0_ref, i1_ref, i2_ref, i3_ref),
                          GROUP_SIZES, GROUP_STARTS):
        ref[...] = lax.slice(sorted_idx, (s0,), (s0 + g,)).reshape(g, 1)

    rm = rm_ref[...]
    om = om_ref[...]
    jj2 = lax.broadcasted_iota(jnp.int32, (2 * C, C), 1)
    oh2 = (jnp.broadcast_to(rank_lanes, (2 * C, C)) == rm).astype(jnp.int32)
    gidx_ref[...] = (jnp.sum(oh2 * jj2, axis=1).reshape(2 * C, 1) + om)


def _sort_and_index(e):
    return pl.pallas_call(
        _rank_body,
        out_shape=tuple(jax.ShapeDtypeStruct((g, 1), jnp.int32)
                        for g in GROUP_SIZES)
                  + (jax.ShapeDtypeStruct((2 * C, 1), jnp.int32),),
    )(e.reshape(1, C), jnp.asarray(_RM).reshape(2 * C, 1),
      jnp.asarray(_OM).reshape(2 * C, 1))


def _gather_body(q_hbm, k_hbm, v_hbm, gidx_hbm, *refs):
    outs = (refs[0:4], refs[4:8], refs[8:12])
    idx_v = refs[12]
    bufs = refs[13:13 + NBUF]
    gsems = refs[13 + NBUF:13 + 2 * NBUF]
    wsems = refs[13 + 2 * NBUF:13 + 3 * NBUF]
    tables = (q_hbm, k_hbm, v_hbm)

    wid = lax.axis_index("s") * NC + lax.axis_index("c")
    pltpu.sync_copy(gidx_hbm.at[pl.ds(wid * IDX_PER_W, IDX_PER_W)], idx_v)

    sched = []
    for t in range(3):
        off = 0
        for g in range(4):
            rpw = ROWS_PER_LEAF_PER_W[g]
            for c in range(rpw // CHUNK):
                sched.append((t, g, off + c * CHUNK, c))
            off += rpw
    S = len(sched)

    def gather(s):
        t, g, ioff, c = sched[s]
        return pltpu.async_copy(
            tables[t].at[idx_v.at[pl.ds(ioff, CHUNK)]],
            bufs[s % NBUF], gsems[s % NBUF])

    def write(s):
        t, g, ioff, c = sched[s]
        row0 = wid * ROWS_PER_LEAF_PER_W[g] + c * CHUNK
        return pltpu.async_copy(
            bufs[s % NBUF], outs[t][g].at[pl.ds(row0, CHUNK)],
            wsems[s % NBUF])

    gh = [None] * S
    wh = [None] * S
    for s in range(min(2, S)):
        gh[s] = gather(s)
    for s in range(S):
        gh[s].wait()
        wh[s] = write(s)
        if s - 1 >= 0:
            wh[s - 1].wait()
        if s + 2 < S:
            gh[s + 2] = gather(s + 2)
    wh[S - 1].wait()


def kernel(q, k, v):
    energy = _energy(q).reshape(C)
    i0, i1, i2, i3, gidx = _sort_and_index(energy)
    gidx = gidx.reshape(2 * C)
    idx_groups = [i.reshape(-1) for i in (i0, i1, i2, i3)]

    out_type = [jax.ShapeDtypeStruct((2 * g, N), jnp.float32)
                for g in GROUP_SIZES] * 3

    gather_call = functools.partial(
        pl.kernel,
        mesh=plsc.VectorSubcoreMesh(core_axis_name="c", subcore_axis_name="s"),
        out_type=out_type,
        scratch_types=(
            [pltpu.VMEM((IDX_PER_W,), jnp.int32)]
            + [pltpu.VMEM((CHUNK, N), jnp.float32)] * NBUF
            + [pltpu.SemaphoreType.DMA] * (2 * NBUF)
        ),
    )(_gather_body)

    flat = gather_call(q.reshape(B * C, N), k.reshape(B * C, N),
                       v.reshape(B * C, N), gidx)
    q_groups = [flat[i].reshape(B, g, N) for i, g in enumerate(GROUP_SIZES)]
    k_groups = [flat[4 + i].reshape(B, g, N) for i, g in enumerate(GROUP_SIZES)]
    v_groups = [flat[8 + i].reshape(B, g, N) for i, g in enumerate(GROUP_SIZES)]
    return tuple(q_groups) + tuple(k_groups) + tuple(v_groups) + tuple(idx_groups)


# speedup vs baseline: 1.0557x; 1.0125x over previous
import functools

import jax
import jax.numpy as jnp
import numpy as np
from jax import lax
from jax.experimental import pallas as pl
from jax.experimental.pallas import tpu as pltpu
from jax.experimental.pallas import tpu_sc as plsc

B, C, N = 2, 1024, 4096
GROUP_SIZES = (128, 128, 256, 512)
GROUP_STARTS = (0, 128, 256, 512)

NC, NS = 2, 16
NW = NC * NS
ROWS_PER_LEAF_PER_W = tuple(2 * g // NW for g in GROUP_SIZES)
IDX_PER_W = 2 * C // NW
CHUNK = 8
NBUF = 3

_RM = np.zeros(2 * C, dtype=np.int32)
_OM = np.zeros(2 * C, dtype=np.int32)
for _w in range(NW):
    _l = 0
    for _g, _s0, _rpw in zip(GROUP_SIZES, GROUP_STARTS, ROWS_PER_LEAF_PER_W):
        for _r in range(_rpw):
            _flat = _w * _rpw + _r
            _b, _j = divmod(_flat, _g)
            _RM[_w * IDX_PER_W + _l] = _s0 + _j
            _OM[_w * IDX_PER_W + _l] = _b * C
            _l += 1


def _energy_body(q_ref, e_ref):
    x = jnp.abs(q_ref[...])
    acc = x[..., 0:128]
    for j in range(1, 32):
        acc = acc + x[..., j * 128:(j + 1) * 128]
    accT = lax.transpose(acc, (0, 2, 1))
    s = accT[:, 0:8, :]
    for v in range(1, 16):
        s = s + accT[:, 8 * v:8 * v + 8, :]
    u = s[:, 0:4, :] + s[:, 4:8, :]
    p = u[:, 0:2, :] + u[:, 2:4, :]
    tot = p[:, 0:1, :] + p[:, 1:2, :]
    inv = np.float32(1.0 / N)
    e_ref[...] = (tot[0:1, :, :] * inv + tot[1:2, :, :] * inv) * np.float32(0.5)


def _energy(q):
    return pl.pallas_call(
        _energy_body,
        grid=(8,),
        in_specs=[pl.BlockSpec((2, 128, 4096), lambda i: (0, i, 0))],
        out_specs=pl.BlockSpec((1, 1, 128), lambda i: (i, 0, 0)),
        out_shape=jax.ShapeDtypeStruct((8, 1, 128), jnp.float32),
    )(q)


def _fused_body(q_ref, rm_ref, om_ref, i0_ref, i1_ref, i2_ref, i3_ref,
                gidx_ref, e_scr):
    i = pl.program_id(0)
    x = jnp.abs(q_ref[...])
    acc = x[..., 0:128]
    for j in range(1, 32):
        acc = acc + x[..., j * 128:(j + 1) * 128]
    accT = lax.transpose(acc, (0, 2, 1))
    s = accT[:, 0:8, :]
    for v in range(1, 16):
        s = s + accT[:, 8 * v:8 * v + 8, :]
    u = s[:, 0:4, :] + s[:, 4:8, :]
    p = u[:, 0:2, :] + u[:, 2:4, :]
    tot = p[:, 0:1, :] + p[:, 1:2, :]
    inv = np.float32(1.0 / N)
    e = (tot[0:1, :, :] * inv + tot[1:2, :, :] * inv) * np.float32(0.5)
    for k in range(8):
        @pl.when(i == k)
        def _store(k=k, e=e):
            e_scr[:, k * 128:(k + 1) * 128] = e[0]

    @pl.when(i == 7)
    def _rank():
        _rank_math(e_scr[...], rm_ref, om_ref,
                   i0_ref, i1_ref, i2_ref, i3_ref, gidx_ref)


def _sort_and_index_fused(q):
    return pl.pallas_call(
        _fused_body,
        grid=(8,),
        in_specs=[
            pl.BlockSpec((2, 128, 4096), lambda i: (0, i, 0)),
            pl.BlockSpec((2 * C, 1), lambda i: (0, 0)),
            pl.BlockSpec((2 * C, 1), lambda i: (0, 0)),
        ],
        out_specs=tuple(pl.BlockSpec((g, 1), lambda i: (0, 0))
                        for g in GROUP_SIZES)
                  + (pl.BlockSpec((2 * C, 1), lambda i: (0, 0)),),
        out_shape=tuple(jax.ShapeDtypeStruct((g, 1), jnp.int32)
                        for g in GROUP_SIZES)
                  + (jax.ShapeDtypeStruct((2 * C, 1), jnp.int32),),
        scratch_shapes=[pltpu.VMEM((1, C), jnp.float32)],
    )(q, jnp.asarray(_RM).reshape(2 * C, 1), jnp.asarray(_OM).reshape(2 * C, 1))


def _rank_math(e, rm_ref, om_ref, i0_ref, i1_ref, i2_ref, i3_ref,
               gidx_ref):
    e_lanes = jnp.broadcast_to(e, (C, C))
    e_rows = lax.broadcast_in_dim(e.reshape(C), (C, C), (0,))
    ii = lax.broadcasted_iota(jnp.int32, (C, C), 0)
    jj = lax.broadcasted_iota(jnp.int32, (C, C), 1)
    before = (e_rows > e_lanes) | ((e_rows == e_lanes) & (ii < jj))
    rank = jnp.sum(before.astype(jnp.int32), axis=0)
    rank_lanes = rank.reshape(1, C)

    rr = lax.broadcasted_iota(jnp.int32, (C, C), 0)
    onehot = (jnp.broadcast_to(rank_lanes, (C, C)) == rr).astype(jnp.int32)
    sorted_idx = jnp.sum(onehot * jj, axis=1)
    for ref, g, s0 in zip((i0_ref, i1_ref, i2_ref, i3_ref),
                          GROUP_SIZES, GROUP_STARTS):
        ref[...] = lax.slice(sorted_idx, (s0,), (s0 + g,)).reshape(g, 1)

    rm = rm_ref[...]
    om = om_ref[...]
    jj2 = lax.broadcasted_iota(jnp.int32, (2 * C, C), 1)
    oh2 = (jnp.broadcast_to(rank_lanes, (2 * C, C)) == rm).astype(jnp.int32)
    gidx_ref[...] = (jnp.sum(oh2 * jj2, axis=1).reshape(2 * C, 1) + om)


def _gather_body(q_hbm, k_hbm, v_hbm, gidx_hbm, *refs):
    outs = (refs[0:4], refs[4:8], refs[8:12])
    idx_v = refs[12]
    bufs = refs[13:13 + NBUF]
    gsems = refs[13 + NBUF:13 + 2 * NBUF]
    wsems = refs[13 + 2 * NBUF:13 + 3 * NBUF]
    tables = (q_hbm, k_hbm, v_hbm)

    wid = lax.axis_index("s") * NC + lax.axis_index("c")
    pltpu.sync_copy(gidx_hbm.at[pl.ds(wid * IDX_PER_W, IDX_PER_W)], idx_v)

    sched = []
    for t in range(3):
        off = 0
        for g in range(4):
            rpw = ROWS_PER_LEAF_PER_W[g]
            for c in range(rpw // CHUNK):
                sched.append((t, g, off + c * CHUNK, c))
            off += rpw
    S = len(sched)

    def gather(s):
        t, g, ioff, c = sched[s]
        return pltpu.async_copy(
            tables[t].at[idx_v.at[pl.ds(ioff, CHUNK)]],
            bufs[s % NBUF], gsems[s % NBUF])

    def write(s):
        t, g, ioff, c = sched[s]
        row0 = wid * ROWS_PER_LEAF_PER_W[g] + c * CHUNK
        return pltpu.async_copy(
            bufs[s % NBUF], outs[t][g].at[pl.ds(row0, CHUNK)],
            wsems[s % NBUF])

    gh = [None] * S
    wh = [None] * S
    for s in range(min(2, S)):
        gh[s] = gather(s)
    for s in range(S):
        gh[s].wait()
        wh[s] = write(s)
        if s - 1 >= 0:
            wh[s - 1].wait()
        if s + 2 < S:
            gh[s + 2] = gather(s + 2)
    wh[S - 1].wait()


def kernel(q, k, v):
    i0, i1, i2, i3, gidx = _sort_and_index_fused(q)
    gidx = gidx.reshape(2 * C)
    idx_groups = [i.reshape(-1) for i in (i0, i1, i2, i3)]

    out_type = [jax.ShapeDtypeStruct((2 * g, N), jnp.float32)
                for g in GROUP_SIZES] * 3

    gather_call = functools.partial(
        pl.kernel,
        mesh=plsc.VectorSubcoreMesh(core_axis_name="c", subcore_axis_name="s"),
        out_type=out_type,
        scratch_types=(
            [pltpu.VMEM((IDX_PER_W,), jnp.int32)]
            + [pltpu.VMEM((CHUNK, N), jnp.float32)] * NBUF
            + [pltpu.SemaphoreType.DMA] * (2 * NBUF)
        ),
    )(_gather_body)

    flat = gather_call(q.reshape(B * C, N), k.reshape(B * C, N),
                       v.reshape(B * C, N), gidx)
    q_groups = [flat[i].reshape(B, g, N) for i, g in enumerate(GROUP_SIZES)]
    k_groups = [flat[4 + i].reshape(B, g, N) for i, g in enumerate(GROUP_SIZES)]
    v_groups = [flat[8 + i].reshape(B, g, N) for i, g in enumerate(GROUP_SIZES)]
    return tuple(q_groups) + tuple(k_groups) + tuple(v_groups) + tuple(idx_groups)


# speedup vs baseline: 1.0580x; 1.0021x over previous
import functools

import jax
import jax.numpy as jnp
import numpy as np
from jax import lax
from jax.experimental import pallas as pl
from jax.experimental.pallas import tpu as pltpu
from jax.experimental.pallas import tpu_sc as plsc

B, C, N = 2, 1024, 4096
GROUP_SIZES = (128, 128, 256, 512)
GROUP_STARTS = (0, 128, 256, 512)

NC, NS = 2, 16
NW = NC * NS
ROWS_PER_LEAF_PER_W = tuple(2 * g // NW for g in GROUP_SIZES)
IDX_PER_W = 2 * C // NW
CHUNK = 8
NBUF = 3

_RM = np.zeros(2 * C, dtype=np.int32)
_OM = np.zeros(2 * C, dtype=np.int32)
for _w in range(NW):
    _l = 0
    for _g, _s0, _rpw in zip(GROUP_SIZES, GROUP_STARTS, ROWS_PER_LEAF_PER_W):
        for _r in range(_rpw):
            _flat = _w * _rpw + _r
            _b, _j = divmod(_flat, _g)
            _RM[_w * IDX_PER_W + _l] = _s0 + _j
            _OM[_w * IDX_PER_W + _l] = _b * C
            _l += 1


def _fused_body(q_ref, rm_ref, om_ref, i0_ref, i1_ref, i2_ref, i3_ref,
                gidx_ref, e_scr):
    i = pl.program_id(0)
    x = jnp.abs(q_ref[...])
    acc = x[..., 0:128]
    for j in range(1, 32):
        acc = acc + x[..., j * 128:(j + 1) * 128]
    accT = lax.transpose(acc, (0, 2, 1))
    s = accT[:, 0:8, :]
    for v in range(1, 16):
        s = s + accT[:, 8 * v:8 * v + 8, :]
    u = s[:, 0:4, :] + s[:, 4:8, :]
    p = u[:, 0:2, :] + u[:, 2:4, :]
    tot = p[:, 0:1, :] + p[:, 1:2, :]
    inv = np.float32(1.0 / N)
    e = (tot[0:1, :, :] * inv + tot[1:2, :, :] * inv) * np.float32(0.5)
    for k in range(8):
        @pl.when(i == k)
        def _store(k=k, e=e):
            e_scr[:, k * 128:(k + 1) * 128] = e[0]

    @pl.when(i == 7)
    def _rank():
        _rank_math(e_scr[...], rm_ref, om_ref,
                   i0_ref, i1_ref, i2_ref, i3_ref, gidx_ref)


def _sort_and_index_fused(q):
    return pl.pallas_call(
        _fused_body,
        grid=(8,),
        in_specs=[
            pl.BlockSpec((2, 128, 4096), lambda i: (0, i, 0)),
            pl.BlockSpec((2 * C, 1), lambda i: (0, 0)),
            pl.BlockSpec((2 * C, 1), lambda i: (0, 0)),
        ],
        out_specs=tuple(pl.BlockSpec((g, 1), lambda i: (0, 0))
                        for g in GROUP_SIZES)
                  + (pl.BlockSpec((2 * C, 1), lambda i: (0, 0)),),
        out_shape=tuple(jax.ShapeDtypeStruct((g, 1), jnp.int32)
                        for g in GROUP_SIZES)
                  + (jax.ShapeDtypeStruct((2 * C, 1), jnp.int32),),
        scratch_shapes=[pltpu.VMEM((1, C), jnp.float32)],
    )(q, jnp.asarray(_RM).reshape(2 * C, 1), jnp.asarray(_OM).reshape(2 * C, 1))


def _rank_math(e, rm_ref, om_ref, i0_ref, i1_ref, i2_ref, i3_ref,
               gidx_ref):
    e_lanes = jnp.broadcast_to(e, (C, C))
    e_rows = lax.broadcast_in_dim(e.reshape(C), (C, C), (0,))
    ii = lax.broadcasted_iota(jnp.int32, (C, C), 0)
    jj = lax.broadcasted_iota(jnp.int32, (C, C), 1)
    before = (e_rows > e_lanes) | ((e_rows == e_lanes) & (ii < jj))
    rank = jnp.sum(before.astype(jnp.int32), axis=0)
    rank_lanes = rank.reshape(1, C)

    rr = lax.broadcasted_iota(jnp.int32, (C, C), 0)
    onehot = (jnp.broadcast_to(rank_lanes, (C, C)) == rr).astype(jnp.int32)
    sorted_idx = jnp.sum(onehot * jj, axis=1)
    for ref, g, s0 in zip((i0_ref, i1_ref, i2_ref, i3_ref),
                          GROUP_SIZES, GROUP_STARTS):
        ref[...] = lax.slice(sorted_idx, (s0,), (s0 + g,)).reshape(g, 1)

    rm = rm_ref[...]
    om = om_ref[...]
    jj2 = lax.broadcasted_iota(jnp.int32, (2 * C, C), 1)
    oh2 = (jnp.broadcast_to(rank_lanes, (2 * C, C)) == rm).astype(jnp.int32)
    gidx_ref[...] = (jnp.sum(oh2 * jj2, axis=1).reshape(2 * C, 1) + om)


def _gather_body(q_hbm, k_hbm, v_hbm, gidx_hbm, *refs):
    outs = (refs[0:4], refs[4:8], refs[8:12])
    idx_v = refs[12]
    bufs = refs[13:13 + NBUF]
    gsems = refs[13 + NBUF:13 + 2 * NBUF]
    wsems = refs[13 + 2 * NBUF:13 + 3 * NBUF]
    tables = (q_hbm, k_hbm, v_hbm)

    wid = lax.axis_index("s") * NC + lax.axis_index("c")
    pltpu.sync_copy(gidx_hbm.at[pl.ds(wid * IDX_PER_W, IDX_PER_W)], idx_v)

    sched = []
    for t in range(3):
        off = 0
        for g in range(4):
            rpw = ROWS_PER_LEAF_PER_W[g]
            for c in range(rpw // CHUNK):
                sched.append((t, g, off + c * CHUNK, c))
            off += rpw
    S = len(sched)

    def gather(s):
        t, g, ioff, c = sched[s]
        return pltpu.async_copy(
            tables[t].at[idx_v.at[pl.ds(ioff, CHUNK)]],
            bufs[s % NBUF], gsems[s % NBUF])

    def write(s):
        t, g, ioff, c = sched[s]
        row0 = wid * ROWS_PER_LEAF_PER_W[g] + c * CHUNK
        return pltpu.async_copy(
            bufs[s % NBUF], outs[t][g].at[pl.ds(row0, CHUNK)],
            wsems[s % NBUF])

    gh = [None] * S
    wh = [None] * S
    for s in range(min(2, S)):
        gh[s] = gather(s)
    for s in range(S):
        gh[s].wait()
        wh[s] = write(s)
        if s - 1 >= 0:
            wh[s - 1].wait()
        if s + 2 < S:
            gh[s + 2] = gather(s + 2)
    wh[S - 1].wait()


def kernel(q, k, v):
    i0, i1, i2, i3, gidx = _sort_and_index_fused(q)
    gidx = gidx.reshape(2 * C)
    idx_groups = [i.reshape(-1) for i in (i0, i1, i2, i3)]

    out_type = [jax.ShapeDtypeStruct((2 * g, N), jnp.float32)
                for g in GROUP_SIZES] * 3

    gather_call = functools.partial(
        pl.kernel,
        mesh=plsc.VectorSubcoreMesh(core_axis_name="c", subcore_axis_name="s"),
        out_type=out_type,
        scratch_types=(
            [pltpu.VMEM((IDX_PER_W,), jnp.int32)]
            + [pltpu.VMEM((CHUNK, N), jnp.float32)] * NBUF
            + [pltpu.SemaphoreType.DMA] * (2 * NBUF)
        ),
    )(_gather_body)

    flat = gather_call(q.reshape(B * C, N), k.reshape(B * C, N),
                       v.reshape(B * C, N), gidx)
    q_groups = [flat[i].reshape(B, g, N) for i, g in enumerate(GROUP_SIZES)]
    k_groups = [flat[4 + i].reshape(B, g, N) for i, g in enumerate(GROUP_SIZES)]
    v_groups = [flat[8 + i].reshape(B, g, N) for i, g in enumerate(GROUP_SIZES)]
    return tuple(q_groups) + tuple(k_groups) + tuple(v_groups) + tuple(idx_groups)
